# baseline (speedup 1.0000x reference)
import functools

import jax
import jax.numpy as jnp
from jax import lax
from jax.experimental import pallas as pl
from jax.experimental.pallas import tpu as pltpu
from jax.experimental.pallas import tpu_sc as plsc

T = 2048
H = 1024
I = 512
E = 8
TOPK = 2
MB = 128
S_PAD = T * TOPK + E * MB
NB = S_PAD // MB
LN = 128

NC, NS, L = 2, 16, 16
NW = NC * NS
TPW = T // NW
CHUNK = 8

_MESH = plsc.VectorSubcoreMesh(core_axis_name="c", subcore_axis_name="s")


def _routing(x, gate_w, corr_bias):
    router_logits = x.astype(jnp.float32) @ gate_w
    routing_weights = jax.nn.softmax(router_logits, axis=1)
    scores = routing_weights + corr_bias.squeeze()
    _, selected_experts = jax.lax.top_k(scores, TOPK)
    w = jnp.take_along_axis(routing_weights, selected_experts, axis=-1)
    w = w / jnp.clip(jnp.sum(w, axis=-1, keepdims=True), 1e-12)
    return router_logits, selected_experts, w


def _positions(selected_experts):
    ce = jnp.sum(jax.nn.one_hot(selected_experts, E, dtype=jnp.int32), axis=1)
    excl = jnp.cumsum(ce, axis=0) - ce
    counts = jnp.sum(ce, axis=0)
    starts = []
    cur = jnp.int32(0)
    for e in range(E):
        starts.append(cur)
        cur = ((cur + counts[e] + MB - 1) // MB) * MB
    starts = jnp.stack(starts)
    rank = jnp.take_along_axis(excl, selected_experts, axis=1)
    position = starts[selected_experts] + rank
    block_to_expert = (
        jnp.searchsorted(starts, jnp.arange(NB, dtype=jnp.int32) * MB, side="right")
        .astype(jnp.int32) - 1)
    return position, block_to_expert


@functools.partial(
    pl.kernel,
    out_type=jax.ShapeDtypeStruct((S_PAD, H), jnp.float32),
    mesh=_MESH,
    scratch_types=[
        pltpu.VMEM((TPW,), jnp.int32),
        pltpu.VMEM((TPW,), jnp.int32),
        pltpu.VMEM((TPW, H), jnp.float32),
        pltpu.SemaphoreType.DMA,
    ],
)
def _dispatch(x_hbm, pos0_hbm, pos1_hbm, xs_hbm, idx0_v, idx1_v, rows_v, sem):
    wid = lax.axis_index("s") * NC + lax.axis_index("c")
    base = wid * TPW
    pltpu.sync_copy(pos0_hbm.at[pl.ds(base, TPW)], idx0_v)
    pltpu.sync_copy(pos1_hbm.at[pl.ds(base, TPW)], idx1_v)
    pltpu.sync_copy(x_hbm.at[pl.ds(base, TPW)], rows_v)
    c0 = pltpu.async_copy(rows_v, xs_hbm.at[idx0_v], sem)
    c1 = pltpu.async_copy(rows_v, xs_hbm.at[idx1_v], sem)
    c0.wait()
    c1.wait()


@functools.partial(
    pl.kernel,
    out_type=jax.ShapeDtypeStruct((T, H), jnp.float32),
    mesh=_MESH,
    scratch_types=[
        pltpu.VMEM((TPW,), jnp.int32),
        pltpu.VMEM((TPW,), jnp.int32),
        pltpu.VMEM((TPW, TOPK, L), jnp.float32),
        pltpu.VMEM((CHUNK, H), jnp.float32),
        pltpu.VMEM((CHUNK, H), jnp.float32),
        pltpu.VMEM((CHUNK, H), jnp.float32),
        pltpu.VMEM((CHUNK, H), jnp.float32),
        pltpu.SemaphoreType.DMA,
    ],
)
def _combine(ys_hbm, pos0_hbm, pos1_hbm, wrep_hbm, sh_hbm, out_hbm,
             idx0_v, idx1_v, w_v, y0_v, y1_v, sh_v, o_v, sem):
    wid = lax.axis_index("s") * NC + lax.axis_index("c")
    base = wid * TPW
    pltpu.sync_copy(pos0_hbm.at[pl.ds(base, TPW)], idx0_v)
    pltpu.sync_copy(pos1_hbm.at[pl.ds(base, TPW)], idx1_v)
    pltpu.sync_copy(wrep_hbm.at[pl.ds(base, TPW)], w_v)
    for ci in range(TPW // CHUNK):
        tb = base + ci * CHUNK
        g0 = pltpu.async_copy(ys_hbm.at[idx0_v.at[pl.ds(ci * CHUNK, CHUNK)]], y0_v, sem)
        g1 = pltpu.async_copy(ys_hbm.at[idx1_v.at[pl.ds(ci * CHUNK, CHUNK)]], y1_v, sem)
        pltpu.sync_copy(sh_hbm.at[pl.ds(tb, CHUNK)], sh_v)
        g0.wait()
        g1.wait()
        for j in range(CHUNK):
            w0 = w_v[ci * CHUNK + j, 0, :]
            w1 = w_v[ci * CHUNK + j, 1, :]

            def cbody(c, _, j=j, w0=w0, w1=w1):
                sl = pl.ds(c * L, L)
                o_v[j, sl] = (w0 * y0_v[j, sl] + w1 * y1_v[j, sl] + sh_v[j, sl])
                return 0

            lax.fori_loop(0, H // L, cbody, 0)
        pltpu.sync_copy(o_v, out_hbm.at[pl.ds(tb, CHUNK)])


def _group_mlp_body(e_ref, xs_ref, wg_ref, wu_ref, wd_ref, ys_ref):
    xb = xs_ref[...].astype(jnp.bfloat16)
    g = jnp.dot(xb, wg_ref[0], preferred_element_type=jnp.float32)
    u = jnp.dot(xb, wu_ref[0], preferred_element_type=jnp.float32)
    h = (g * jax.nn.sigmoid(g) * u).astype(jnp.bfloat16)
    ys_ref[...] = jnp.dot(h, wd_ref[0], preferred_element_type=jnp.float32)


def _shared_body(x_ref, sg_ref, su_ref, sd_ref, out_ref):
    xb = x_ref[...].astype(jnp.bfloat16)
    g = jnp.dot(xb, sg_ref[...], preferred_element_type=jnp.float32)
    u = jnp.dot(xb, su_ref[...], preferred_element_type=jnp.float32)
    h = (g * jax.nn.sigmoid(g) * u).astype(jnp.bfloat16)
    out_ref[...] = jnp.dot(h, sd_ref[...], preferred_element_type=jnp.float32)


def kernel(hidden_states, gate_w, corr_bias, Wg, Wu, Wd, Sg, Su, Sd):
    b, s, h = hidden_states.shape
    x = hidden_states.reshape(T, H)

    router_logits, selected_experts, w = _routing(x, gate_w, corr_bias)
    position, block_to_expert = _positions(selected_experts)
    pos0 = position[:, 0].astype(jnp.int32)
    pos1 = position[:, 1].astype(jnp.int32)
    w_rep = jnp.broadcast_to(w[:, :, None], (T, TOPK, L)).astype(jnp.float32)

    xs = _dispatch(x, pos0, pos1)

    wg8 = Wg.astype(jnp.bfloat16)
    wu8 = Wu.astype(jnp.bfloat16)
    wd8 = Wd.astype(jnp.bfloat16)
    grid_spec = pltpu.PrefetchScalarGridSpec(
        num_scalar_prefetch=1,
        grid=(NB,),
        in_specs=[
            pl.BlockSpec((MB, H), lambda m, e_ref: (m, 0)),
            pl.BlockSpec((1, H, I), lambda m, e_ref: (e_ref[m], 0, 0)),
            pl.BlockSpec((1, H, I), lambda m, e_ref: (e_ref[m], 0, 0)),
            pl.BlockSpec((1, I, H), lambda m, e_ref: (e_ref[m], 0, 0)),
        ],
        out_specs=pl.BlockSpec((MB, H), lambda m, e_ref: (m, 0)),
    )
    ys = pl.pallas_call(
        _group_mlp_body,
        grid_spec=grid_spec,
        out_shape=jax.ShapeDtypeStruct((S_PAD, H), jnp.float32),
    )(block_to_expert, xs, wg8, wu8, wd8)

    BT = 1024
    shared_out = pl.pallas_call(
        _shared_body,
        grid=(T // BT,),
        in_specs=[
            pl.BlockSpec((BT, H), lambda i: (i, 0)),
            pl.BlockSpec((H, H), lambda i: (0, 0)),
            pl.BlockSpec((H, H), lambda i: (0, 0)),
            pl.BlockSpec((H, H), lambda i: (0, 0)),
        ],
        out_specs=pl.BlockSpec((BT, H), lambda i: (i, 0)),
        out_shape=jax.ShapeDtypeStruct((T, H), jnp.float32),
    )(x, Sg.astype(jnp.bfloat16), Su.astype(jnp.bfloat16), Sd.astype(jnp.bfloat16))

    out = _combine(ys, pos0, pos1, w_rep, shared_out)

    return out.reshape(b, s, h), router_logits
